# baseline (speedup 1.0000x reference)
import functools

import jax
import jax.numpy as jnp
from jax import lax
from jax.experimental import pallas as pl
from jax.experimental.pallas import tpu as pltpu
from jax.experimental.pallas import tpu_sc as plsc

N_NEG = 20
N_ROWS = N_NEG + 1
B = 16384
NC = 2
NS = 16
NW = NC * NS
B_PER_W = B // NW
L = 16


def _sc_denom_product(x, pos, neg):
    mesh = plsc.VectorSubcoreMesh(core_axis_name="c", subcore_axis_name="s")

    @functools.partial(
        pl.kernel,
        mesh=mesh,
        out_type=jax.ShapeDtypeStruct((B,), jnp.float32),
        scratch_types=[
            pltpu.VMEM((N_ROWS * B_PER_W,), jnp.int32),
            pltpu.VMEM((N_ROWS * B_PER_W,), jnp.float32),
            pltpu.VMEM((B_PER_W,), jnp.float32),
            pltpu.SemaphoreType.DMA,
            pltpu.SemaphoreType.DMA,
        ],
    )
    def k(x_hbm, pos_hbm, neg_hbm, out_hbm, idx_v, g_v, d_v, sem_i, sem_g):
        wid = lax.axis_index("s") * NC + lax.axis_index("c")
        base = wid * B_PER_W

        cps = [pltpu.async_copy(pos_hbm.at[pl.ds(base, B_PER_W)],
                                idx_v.at[pl.ds(0, B_PER_W)], sem_i)]
        for i in range(N_NEG):
            cps.append(pltpu.async_copy(
                neg_hbm.at[i, pl.ds(base, B_PER_W)],
                idx_v.at[pl.ds((i + 1) * B_PER_W, B_PER_W)], sem_i))

        gps = []
        for i in range(N_ROWS):
            cps[i].wait()
            gps.append(pltpu.async_copy(
                x_hbm.at[idx_v.at[pl.ds(i * B_PER_W, B_PER_W)]],
                g_v.at[pl.ds(i * B_PER_W, B_PER_W)], sem_g))

        gps[0].wait()

        @pl.loop(0, B_PER_W, step=L)
        def _(jv):
            v = g_v[pl.ds(jv, L)]
            d_v[pl.ds(jv, L)] = 1.0 + jnp.exp(-v)

        for i in range(1, N_ROWS):
            gps[i].wait()

            @pl.loop(0, B_PER_W, step=L)
            def _(jv, i=i):
                vi = g_v[pl.ds(i * B_PER_W + jv, L)]
                d_v[pl.ds(jv, L)] = d_v[pl.ds(jv, L)] * (1.0 + jnp.exp(vi))

        pltpu.sync_copy(d_v, out_hbm.at[pl.ds(base, B_PER_W)])

    return k(x, pos, neg)


def _tc_log(d):
    def body(d_ref, o_ref):
        o_ref[...] = jnp.log(d_ref[...])

    out = pl.pallas_call(
        body,
        out_shape=jax.ShapeDtypeStruct((B // 128, 128), jnp.float32),
    )(d.reshape(B // 128, 128))
    return out.reshape(B)


def kernel(x, positiveItem, negativeItem):
    pos = positiveItem.astype(jnp.int32)
    neg = negativeItem.astype(jnp.int32)
    d = _sc_denom_product(x, pos, neg)
    return _tc_log(d)


# speedup vs baseline: 1.2220x; 1.2220x over previous
import functools

import jax
import jax.numpy as jnp
from jax import lax
from jax.experimental import pallas as pl
from jax.experimental.pallas import tpu as pltpu
from jax.experimental.pallas import tpu_sc as plsc

N_NEG = 20
N_ROWS = N_NEG + 1
B = 16384
NC = 2
NS = 16
NW = NC * NS
B_PER_W = B // NW
L = 16


def _sc_denom_product(x, pos, neg):
    mesh = plsc.VectorSubcoreMesh(core_axis_name="c", subcore_axis_name="s")

    @functools.partial(
        pl.kernel,
        mesh=mesh,
        out_type=jax.ShapeDtypeStruct((B,), jnp.float32),
        scratch_types=[
            pltpu.VMEM((N_ROWS * B_PER_W,), jnp.int32),
            pltpu.VMEM((N_ROWS * B_PER_W,), jnp.float32),
            pltpu.VMEM((B_PER_W,), jnp.float32),
            pltpu.SemaphoreType.DMA,
            pltpu.SemaphoreType.DMA,
        ],
    )
    def k(x_hbm, pos_hbm, neg_hbm, out_hbm, idx_v, g_v, d_v, sem_i, sem_g):
        wid = lax.axis_index("s") * NC + lax.axis_index("c")
        base = wid * B_PER_W

        cps = [pltpu.async_copy(pos_hbm.at[pl.ds(base, B_PER_W)],
                                idx_v.at[pl.ds(0, B_PER_W)], sem_i)]
        for i in range(N_NEG):
            cps.append(pltpu.async_copy(
                neg_hbm.at[i, pl.ds(base, B_PER_W)],
                idx_v.at[pl.ds((i + 1) * B_PER_W, B_PER_W)], sem_i))

        gps = []
        for i in range(N_ROWS):
            cps[i].wait()
            gps.append(pltpu.async_copy(
                x_hbm.at[idx_v.at[pl.ds(i * B_PER_W, B_PER_W)]],
                g_v.at[pl.ds(i * B_PER_W, B_PER_W)], sem_g))
        for gp in gps:
            gp.wait()

        @pl.loop(0, B_PER_W, step=L)
        def _(jv):
            v = g_v[pl.ds(jv, L)]
            d = 1.0 + jnp.exp(-v)
            for i in range(1, N_ROWS):
                vi = g_v[pl.ds(i * B_PER_W + jv, L)]
                d = d * (1.0 + jnp.exp(vi))
            d_v[pl.ds(jv, L)] = d

        pltpu.sync_copy(d_v, out_hbm.at[pl.ds(base, B_PER_W)])

    return k(x, pos, neg)


def _tc_log(d):
    def body(d_ref, o_ref):
        o_ref[...] = jnp.log(d_ref[...])

    out = pl.pallas_call(
        body,
        out_shape=jax.ShapeDtypeStruct((B // 128, 128), jnp.float32),
    )(d.reshape(B // 128, 128))
    return out.reshape(B)


def kernel(x, positiveItem, negativeItem):
    pos = positiveItem.astype(jnp.int32)
    neg = negativeItem.astype(jnp.int32)
    d = _sc_denom_product(x, pos, neg)
    return _tc_log(d)
